# initial kernel scaffold (unmeasured)
import jax
import jax.numpy as jnp
from jax import lax
from jax.experimental import pallas as pl
from jax.experimental.pallas import tpu as pltpu

N_DEV = 4
M_LOC = 1024
HALF = 512
K = 4096
N_LOC = 2048


def kernel(x, w_mat, scale_x, scale_w):
    me = lax.axis_index("i")
    xq = x.astype(jnp.float8_e4m3fn).reshape(2, HALF, K)
    w_loc = lax.dynamic_slice(
        w_mat, (0, me * N_LOC), (K, N_LOC)
    ).astype(jnp.float8_e5m2)

    def body(x_ref, w_ref, sx_ref, sw_ref, out_ref, bufL, bufR, bufO,
             send_sems, recv_sems):
        me = lax.axis_index("i")
        left = lax.rem(me + N_DEV - 1, N_DEV)
        right = lax.rem(me + 1, N_DEV)
        opp = lax.rem(me + 2, N_DEV)

        barrier_sem = pltpu.get_barrier_semaphore()
        for nbr in (left, right):
            pl.semaphore_signal(barrier_sem, inc=1, device_id=(nbr,),
                                device_id_type=pl.DeviceIdType.MESH)
        pl.semaphore_wait(barrier_sem, 2)

        scale = sx_ref[0] * sw_ref[0]

        def mm(a):
            acc = lax.dot_general(
                a, w_ref[...],
                (((1,), (0,)), ((), ())),
                preferred_element_type=jnp.float32,
            )
            return acc * scale

        r0 = pltpu.make_async_remote_copy(
            src_ref=x_ref, dst_ref=bufL,
            send_sem=send_sems.at[0], recv_sem=recv_sems.at[0],
            device_id=(right,), device_id_type=pl.DeviceIdType.MESH)
        l0 = pltpu.make_async_remote_copy(
            src_ref=x_ref, dst_ref=bufR,
            send_sem=send_sems.at[1], recv_sem=recv_sems.at[1],
            device_id=(left,), device_id_type=pl.DeviceIdType.MESH)
        r0.start()
        l0.start()

        for h in range(2):
            out_ref[pl.ds(me * M_LOC + h * HALF, HALF), :] = mm(x_ref[h])

        r0.wait_recv()
        f_r = pltpu.make_async_remote_copy(
            src_ref=bufL.at[0], dst_ref=bufO.at[0],
            send_sem=send_sems.at[2], recv_sem=recv_sems.at[2],
            device_id=(right,), device_id_type=pl.DeviceIdType.MESH)
        f_r.start()
        l0.wait_recv()
        f_l = pltpu.make_async_remote_copy(
            src_ref=bufR.at[1], dst_ref=bufO.at[1],
            send_sem=send_sems.at[3], recv_sem=recv_sems.at[3],
            device_id=(left,), device_id_type=pl.DeviceIdType.MESH)
        f_l.start()

        for h in range(2):
            out_ref[pl.ds(left * M_LOC + h * HALF, HALF), :] = mm(bufL[h])
        for h in range(2):
            out_ref[pl.ds(right * M_LOC + h * HALF, HALF), :] = mm(bufR[h])

        f_r.wait_recv()
        f_l.wait_recv()
        for h in range(2):
            out_ref[pl.ds(opp * M_LOC + h * HALF, HALF), :] = mm(bufO[h])

        r0.wait_send()
        l0.wait_send()
        f_r.wait_send()
        f_l.wait_send()

    fp8buf = pltpu.VMEM((2, HALF, K), jnp.float8_e4m3fn)
    return pl.pallas_call(
        body,
        out_shape=jax.ShapeDtypeStruct((N_DEV * M_LOC, N_LOC), jnp.float32),
        in_specs=[
            pl.BlockSpec(memory_space=pltpu.VMEM),
            pl.BlockSpec(memory_space=pltpu.VMEM),
            pl.BlockSpec(memory_space=pltpu.SMEM),
            pl.BlockSpec(memory_space=pltpu.SMEM),
        ],
        out_specs=pl.BlockSpec(memory_space=pltpu.VMEM),
        scratch_shapes=[
            fp8buf, fp8buf, fp8buf,
            pltpu.SemaphoreType.DMA((4,)),
            pltpu.SemaphoreType.DMA((4,)),
        ],
        compiler_params=pltpu.CompilerParams(collective_id=0),
    )(xq, w_loc, scale_x, scale_w)


# baseline (device time: 148354 ns/iter reference)
import jax
import jax.numpy as jnp
from jax import lax
from jax.experimental import pallas as pl
from jax.experimental.pallas import tpu as pltpu

N_DEV = 4
M_LOC = 1024
HALF = 512
K = 4096
N_LOC = 2048


def kernel(x, w_mat, scale_x, scale_w):
    me = lax.axis_index("i")
    xq = x.astype(jnp.float8_e4m3fn).reshape(2, HALF, K)
    w_loc = lax.dynamic_slice(
        w_mat, (0, me * N_LOC), (K, N_LOC)
    ).astype(jnp.float8_e5m2)

    def body(x_ref, w_ref, sx_ref, sw_ref, out_ref, bufL, bufR, bufO,
             send_sems, recv_sems):
        me = lax.axis_index("i")
        left = lax.rem(me + N_DEV - 1, N_DEV)
        right = lax.rem(me + 1, N_DEV)
        opp = lax.rem(me + 2, N_DEV)

        barrier_sem = pltpu.get_barrier_semaphore()
        for nbr in (left, right):
            pl.semaphore_signal(barrier_sem, inc=1, device_id=(nbr,),
                                device_id_type=pl.DeviceIdType.MESH)
        pl.semaphore_wait(barrier_sem, 2)

        scale = sx_ref[0] * sw_ref[0]

        def mm(a):
            acc = lax.dot_general(
                a, w_ref[...],
                (((1,), (0,)), ((), ())),
                preferred_element_type=jnp.float32,
            )
            return acc * scale

        r0 = pltpu.make_async_remote_copy(
            src_ref=x_ref, dst_ref=bufL,
            send_sem=send_sems.at[0], recv_sem=recv_sems.at[0],
            device_id=(right,), device_id_type=pl.DeviceIdType.MESH)
        l0 = pltpu.make_async_remote_copy(
            src_ref=x_ref, dst_ref=bufR,
            send_sem=send_sems.at[1], recv_sem=recv_sems.at[1],
            device_id=(left,), device_id_type=pl.DeviceIdType.MESH)
        r0.start()
        l0.start()

        for h in range(2):
            out_ref[pl.ds(me * M_LOC + h * HALF, HALF), :] = mm(x_ref[h])

        r0.wait_recv()
        f_r = pltpu.make_async_remote_copy(
            src_ref=bufL.at[0], dst_ref=bufO.at[0],
            send_sem=send_sems.at[2], recv_sem=recv_sems.at[2],
            device_id=(right,), device_id_type=pl.DeviceIdType.MESH)
        f_r.start()
        l0.wait_recv()
        f_l = pltpu.make_async_remote_copy(
            src_ref=bufR.at[1], dst_ref=bufO.at[1],
            send_sem=send_sems.at[3], recv_sem=recv_sems.at[3],
            device_id=(left,), device_id_type=pl.DeviceIdType.MESH)
        f_l.start()

        for h in range(2):
            out_ref[pl.ds(left * M_LOC + h * HALF, HALF), :] = mm(bufL[h])
        for h in range(2):
            out_ref[pl.ds(right * M_LOC + h * HALF, HALF), :] = mm(bufR[h])

        f_r.wait_recv()
        f_l.wait_recv()
        for h in range(2):
            out_ref[pl.ds(opp * M_LOC + h * HALF, HALF), :] = mm(bufO[h])

        r0.wait_send()
        l0.wait_send()
        f_r.wait_send()
        f_l.wait_send()

    fp8buf = pltpu.VMEM((2, HALF, K), jnp.float8_e4m3fn)
    return pl.pallas_call(
        body,
        out_shape=jax.ShapeDtypeStruct((N_DEV * M_LOC, N_LOC), jnp.float32),
        in_specs=[
            pl.BlockSpec(memory_space=pltpu.VMEM),
            pl.BlockSpec(memory_space=pltpu.VMEM),
            pl.BlockSpec(memory_space=pltpu.SMEM),
            pl.BlockSpec(memory_space=pltpu.SMEM),
        ],
        out_specs=pl.BlockSpec(memory_space=pltpu.VMEM),
        scratch_shapes=[
            fp8buf, fp8buf, fp8buf,
            pltpu.SemaphoreType.DMA((4,)),
            pltpu.SemaphoreType.DMA((4,)),
        ],
        compiler_params=pltpu.CompilerParams(
            collective_id=0,
            vmem_limit_bytes=100 * 1024 * 1024,
        ),
    )(xq, w_loc, scale_x, scale_w)


# device time: 109379 ns/iter; 1.3563x vs baseline; 1.3563x over previous
import jax
import jax.numpy as jnp
from jax import lax
from jax.experimental import pallas as pl
from jax.experimental.pallas import tpu as pltpu

N_DEV = 4
M_LOC = 1024
HALF = 512
K = 4096
KC = 512
N_LOC = 2048


def kernel(x, w_mat, scale_x, scale_w):
    def body(x_hbm, w_hbm, sx_ref, sw_ref, out_hbm,
             xq, bufL, bufR, bufO, wq, wtmp, xtmp, stage,
             send_sems, recv_sems, wdma_sems, xdma_sem, out_sems):
        me = lax.axis_index("i")
        left = lax.rem(me + N_DEV - 1, N_DEV)
        right = lax.rem(me + 1, N_DEV)
        opp = lax.rem(me + 2, N_DEV)

        def rdma(src, dst, i, dev):
            return pltpu.make_async_remote_copy(
                src_ref=src, dst_ref=dst,
                send_sem=send_sems.at[i], recv_sem=recv_sems.at[i],
                device_id=(dev,), device_id_type=pl.DeviceIdType.MESH)

        def xcopy(h):
            return pltpu.make_async_copy(
                x_hbm.at[pl.ds(h * HALF, HALF), :], xtmp, xdma_sem)

        cx = xcopy(0)
        cx.start()

        barrier_sem = pltpu.get_barrier_semaphore()
        for nbr in (left, right):
            pl.semaphore_signal(barrier_sem, inc=1, device_id=(nbr,),
                                device_id_type=pl.DeviceIdType.MESH)
        pl.semaphore_wait(barrier_sem, 2)

        cx.wait()
        xq[0] = xtmp[...].astype(jnp.float8_e4m3fn)

        r0a = rdma(xq.at[0], bufL.at[0], 0, right)
        l0a = rdma(xq.at[0], bufR.at[0], 2, left)
        r0a.start()
        l0a.start()

        cx = xcopy(1)
        cx.start()
        cx.wait()
        xq[1] = xtmp[...].astype(jnp.float8_e4m3fn)
        r0b = rdma(xq.at[1], bufL.at[1], 1, right)
        l0b = rdma(xq.at[1], bufR.at[1], 3, left)
        r0b.start()
        l0b.start()

        def wcopy(k, slot):
            return pltpu.make_async_copy(
                w_hbm.at[pl.ds(k * KC, KC), pl.ds(me * N_LOC, N_LOC)],
                wtmp.at[slot], wdma_sems.at[slot])

        wcopy(0, 0).start()
        wcopy(1, 1).start()
        for k in range(K // KC):
            wcopy(k, k % 2).wait()
            wq[pl.ds(k * KC, KC), :] = wtmp[k % 2].astype(jnp.float8_e5m2)
            if k + 2 < K // KC:
                wcopy(k + 2, k % 2).start()

        scale = sx_ref[0] * sw_ref[0]

        def mm(a):
            acc = lax.dot_general(
                a, wq[...],
                (((1,), (0,)), ((), ())),
                preferred_element_type=jnp.float32,
            )
            return acc * scale

        pending = [None, None]
        nproduced = [0]

        def produce(origin, h, src):
            slot = nproduced[0] % 2
            nproduced[0] += 1
            if pending[slot] is not None:
                pending[slot].wait()
            stage[slot] = mm(src)
            cp = pltpu.make_async_copy(
                stage.at[slot],
                out_hbm.at[pl.ds(origin * M_LOC + h * HALF, HALF), :],
                out_sems.at[slot])
            cp.start()
            pending[slot] = cp

        produce(me, 0, xq[0])
        produce(me, 1, xq[1])

        r0a.wait_recv()
        f_r = rdma(bufL.at[0], bufO.at[0], 4, right)
        f_r.start()
        l0b.wait_recv()
        f_l = rdma(bufR.at[1], bufO.at[1], 5, left)
        f_l.start()

        r0b.wait_recv()
        produce(left, 0, bufL[0])
        produce(left, 1, bufL[1])
        l0a.wait_recv()
        produce(right, 0, bufR[0])
        produce(right, 1, bufR[1])

        f_r.wait_recv()
        produce(opp, 0, bufO[0])
        f_l.wait_recv()
        produce(opp, 1, bufO[1])

        r0a.wait_send()
        r0b.wait_send()
        l0a.wait_send()
        l0b.wait_send()
        f_r.wait_send()
        f_l.wait_send()
        pending[0].wait()
        pending[1].wait()

    halves = pltpu.VMEM((2, HALF, K), jnp.float8_e4m3fn)
    return pl.pallas_call(
        body,
        out_shape=jax.ShapeDtypeStruct((N_DEV * M_LOC, N_LOC), jnp.float32),
        in_specs=[
            pl.BlockSpec(memory_space=pl.ANY),
            pl.BlockSpec(memory_space=pl.ANY),
            pl.BlockSpec(memory_space=pltpu.SMEM),
            pl.BlockSpec(memory_space=pltpu.SMEM),
        ],
        out_specs=pl.BlockSpec(memory_space=pl.ANY),
        scratch_shapes=[
            halves,
            halves, halves, halves,
            pltpu.VMEM((K, N_LOC), jnp.float8_e5m2),
            pltpu.VMEM((2, KC, N_LOC), jnp.float32),
            pltpu.VMEM((HALF, K), jnp.float32),
            pltpu.VMEM((2, HALF, N_LOC), jnp.float32),
            pltpu.SemaphoreType.DMA((6,)),
            pltpu.SemaphoreType.DMA((6,)),
            pltpu.SemaphoreType.DMA((2,)),
            pltpu.SemaphoreType.DMA,
            pltpu.SemaphoreType.DMA((2,)),
        ],
        compiler_params=pltpu.CompilerParams(
            collective_id=0,
            vmem_limit_bytes=100 * 1024 * 1024,
        ),
    )(x, w_mat, scale_x, scale_w)


# device time: 102857 ns/iter; 1.4423x vs baseline; 1.0634x over previous
import jax
import jax.numpy as jnp
from jax import lax
from jax.experimental import pallas as pl
from jax.experimental.pallas import tpu as pltpu

N_DEV = 4
M_LOC = 1024
HALF = 512
Q = 256
K = 4096
KC = 512
N_LOC = 2048


def kernel(x, w_mat, scale_x, scale_w):
    def body(x_hbm, w_hbm, sx_ref, sw_ref, out_hbm,
             xq, bufL, bufR, bufO, wq, wtmp, xtmp, stage,
             send_sems, recv_sems, wdma_sems, xdma_sems, out_sems):
        me = lax.axis_index("i")
        left = lax.rem(me + N_DEV - 1, N_DEV)
        right = lax.rem(me + 1, N_DEV)
        opp = lax.rem(me + 2, N_DEV)

        def quarter(ref, q):
            return ref.at[pl.ds(q * Q, Q), :]

        def rdma(src, dst, i, dev):
            return pltpu.make_async_remote_copy(
                src_ref=src, dst_ref=dst,
                send_sem=send_sems.at[i], recv_sem=recv_sems.at[i],
                device_id=(dev,), device_id_type=pl.DeviceIdType.MESH)

        def hop0(q):
            return (rdma(quarter(xq, q), quarter(bufL, q), q, right),
                    rdma(quarter(xq, q), quarter(bufR, q), 4 + q, left))

        fwd_r = [rdma(quarter(bufL, j), quarter(bufO, j), 8 + j, right)
                 for j in range(2)]
        fwd_l = [rdma(quarter(bufR, 2 + j), quarter(bufO, 2 + j), 10 + j, left)
                 for j in range(2)]

        def xcopy(q, slot):
            return pltpu.make_async_copy(
                x_hbm.at[pl.ds(q * Q, Q), :], xtmp.at[slot],
                xdma_sems.at[slot])

        xcopy(0, 0).start()
        xcopy(1, 1).start()

        barrier_sem = pltpu.get_barrier_semaphore()
        for nbr in (left, right):
            pl.semaphore_signal(barrier_sem, inc=1, device_id=(nbr,),
                                device_id_type=pl.DeviceIdType.MESH)
        pl.semaphore_wait(barrier_sem, 2)

        for q in range(4):
            xcopy(q, q % 2).wait()
            xq[pl.ds(q * Q, Q), :] = xtmp[q % 2].astype(jnp.float8_e4m3fn)
            if q + 2 < 4:
                xcopy(q + 2, q % 2).start()
            ra, la = hop0(q)
            ra.start()
            la.start()

        def wcopy(k, slot):
            return pltpu.make_async_copy(
                w_hbm.at[pl.ds(k * KC, KC), pl.ds(me * N_LOC, N_LOC)],
                wtmp.at[slot], wdma_sems.at[slot])

        wcopy(0, 0).start()
        wcopy(1, 1).start()
        for k in range(K // KC):
            wcopy(k, k % 2).wait()
            wq[pl.ds(k * KC, KC), :] = wtmp[k % 2].astype(jnp.float8_e5m2)
            if k + 2 < K // KC:
                wcopy(k + 2, k % 2).start()

        scale = sx_ref[0] * sw_ref[0]

        def mm(src, rows, row0):
            acc = lax.dot_general(
                src[pl.ds(row0, rows), :], wq[...],
                (((1,), (0,)), ((), ())),
                preferred_element_type=jnp.float32,
            )
            return acc * scale

        pending = [None, None]
        state = [0]

        def produce(origin, row0, rows, src):
            slot = state[0] % 2
            state[0] += 1
            if pending[slot] is not None:
                pending[slot].wait()
            stage[slot, pl.ds(0, rows), :] = mm(src, rows, row0)
            cp = pltpu.make_async_copy(
                stage.at[slot, pl.ds(0, rows), :],
                out_hbm.at[pl.ds(origin * M_LOC + row0, rows), :],
                out_sems.at[slot])
            cp.start()
            pending[slot] = cp

        produce(me, 0, HALF, xq)
        produce(me, HALF, HALF, xq)

        recv_r = [rdma(quarter(xq, q), quarter(bufL, q), q, right)
                  for q in range(4)]
        recv_l = [rdma(quarter(xq, q), quarter(bufR, q), 4 + q, left)
                  for q in range(4)]

        recv_r[0].wait_recv()
        fwd_r[0].start()
        recv_r[1].wait_recv()
        fwd_r[1].start()
        recv_l[0].wait_recv()
        recv_l[1].wait_recv()
        produce(right, 0, HALF, bufR)
        recv_l[2].wait_recv()
        fwd_l[0].start()
        recv_l[3].wait_recv()
        fwd_l[1].start()
        produce(right, HALF, HALF, bufR)
        recv_r[2].wait_recv()
        recv_r[3].wait_recv()
        produce(left, 0, HALF, bufL)
        produce(left, HALF, HALF, bufL)

        fwd_r[0].wait_recv()
        produce(opp, 0, Q, bufO)
        fwd_l[0].wait_recv()
        produce(opp, 2 * Q, Q, bufO)
        fwd_r[1].wait_recv()
        produce(opp, Q, Q, bufO)
        fwd_l[1].wait_recv()
        produce(opp, 3 * Q, Q, bufO)

        for q in range(4):
            recv_r[q].wait_send()
            recv_l[q].wait_send()
        fwd_r[0].wait_send()
        fwd_r[1].wait_send()
        fwd_l[0].wait_send()
        fwd_l[1].wait_send()
        pending[0].wait()
        pending[1].wait()

    fp8full = pltpu.VMEM((M_LOC, K), jnp.float8_e4m3fn)
    return pl.pallas_call(
        body,
        out_shape=jax.ShapeDtypeStruct((N_DEV * M_LOC, N_LOC), jnp.float32),
        in_specs=[
            pl.BlockSpec(memory_space=pl.ANY),
            pl.BlockSpec(memory_space=pl.ANY),
            pl.BlockSpec(memory_space=pltpu.SMEM),
            pl.BlockSpec(memory_space=pltpu.SMEM),
        ],
        out_specs=pl.BlockSpec(memory_space=pl.ANY),
        scratch_shapes=[
            fp8full,
            fp8full, fp8full, fp8full,
            pltpu.VMEM((K, N_LOC), jnp.float8_e5m2),
            pltpu.VMEM((2, KC, N_LOC), jnp.float32),
            pltpu.VMEM((2, Q, K), jnp.float32),
            pltpu.VMEM((2, HALF, N_LOC), jnp.float32),
            pltpu.SemaphoreType.DMA((12,)),
            pltpu.SemaphoreType.DMA((12,)),
            pltpu.SemaphoreType.DMA((2,)),
            pltpu.SemaphoreType.DMA((2,)),
            pltpu.SemaphoreType.DMA((2,)),
        ],
        compiler_params=pltpu.CompilerParams(
            collective_id=0,
            vmem_limit_bytes=100 * 1024 * 1024,
        ),
    )(x, w_mat, scale_x, scale_w)
